# Initial kernel scaffold; baseline (speedup 1.0000x reference)
import jax
import jax.numpy as jnp
from jax.experimental import pallas as pl


def kernel(embedding_weight, edge_index):
    raise NotImplementedError("write your pallas kernel here")


# speedup vs baseline: 11.9123x; 11.9123x over previous
import functools

import jax
import jax.numpy as jnp
from jax import lax
from jax.experimental import pallas as pl
from jax.experimental.pallas import tpu as pltpu
from jax.experimental.pallas import tpu_sc as plsc

N = 10000
D = 128
E = 320000
NLAYERS = 3
NC = 2
NS = 16
NW = NC * NS
CHUNK = 128
CPT = 80
EPAD = NW * CPT * CHUNK
NPAD = 10240
RPT = NPAD // NS

_mesh = plsc.VectorSubcoreMesh(
    core_axis_name="c", subcore_axis_name="s", num_cores=NC, num_subcores=NS)


def _deg_body(colp_hbm, ones_hbm, zeros_hbm, out_hbm, col_v, ones_v, acc_sh):
    cid = lax.axis_index("c")
    sid = lax.axis_index("s")
    wid = sid * NC + cid
    r0 = sid * RPT
    pltpu.sync_copy(zeros_hbm.at[pl.ds(r0, RPT)], acc_sh.at[pl.ds(r0, RPT)])
    pltpu.sync_copy(colp_hbm.at[wid], col_v)
    pltpu.sync_copy(ones_hbm, ones_v)
    plsc.subcore_barrier()

    def body(j, carry):
        pltpu.sync_copy(ones_v, acc_sh.at[col_v.at[j]], add=True)
        return carry

    lax.fori_loop(0, CPT, body, 0)
    plsc.subcore_barrier()
    pltpu.sync_copy(acc_sh.at[pl.ds(r0, RPT)], out_hbm.at[cid, pl.ds(r0, RPT)])


_deg_call = pl.kernel(
    _deg_body,
    out_type=jax.ShapeDtypeStruct((NC, NPAD, D), jnp.float32),
    mesh=_mesh,
    scratch_types=[
        pltpu.VMEM((CPT, CHUNK), jnp.int32),
        pltpu.VMEM((CHUNK, D), jnp.float32),
        pltpu.VMEM_SHARED((NPAD, D), jnp.float32),
    ],
)


def _prop_body(u_hbm, rowp_hbm, colp_hbm, zeros_hbm, out_hbm,
               row_v, col_v, gbuf, acc_sh, sem):
    cid = lax.axis_index("c")
    sid = lax.axis_index("s")
    wid = sid * NC + cid
    r0 = sid * RPT
    pltpu.sync_copy(zeros_hbm.at[pl.ds(r0, RPT)], acc_sh.at[pl.ds(r0, RPT)])
    pltpu.sync_copy(rowp_hbm.at[wid], row_v)
    pltpu.sync_copy(colp_hbm.at[wid], col_v)
    plsc.subcore_barrier()

    def body(j, carry):
        pltpu.async_copy(u_hbm.at[row_v.at[j]], gbuf, sem).wait()
        pltpu.sync_copy(gbuf, acc_sh.at[col_v.at[j]], add=True)
        return carry

    lax.fori_loop(0, CPT, body, 0)
    plsc.subcore_barrier()
    pltpu.sync_copy(acc_sh.at[pl.ds(r0, RPT)], out_hbm.at[cid, pl.ds(r0, RPT)])


_prop_call = pl.kernel(
    _prop_body,
    out_type=jax.ShapeDtypeStruct((NC, NPAD, D), jnp.float32),
    mesh=_mesh,
    scratch_types=[
        pltpu.VMEM((CPT, CHUNK), jnp.int32),
        pltpu.VMEM((CPT, CHUNK), jnp.int32),
        pltpu.VMEM((CHUNK, D), jnp.float32),
        pltpu.VMEM_SHARED((NPAD, D), jnp.float32),
        pltpu.SemaphoreType.DMA,
    ],
)


BLK = 512


def _prologue_tc(d0_ref, d1_ref, x0_ref, u0_ref, s2_ref):
    deg = d0_ref[:, 0:1] + d1_ref[:, 0:1]
    s = jnp.where(deg > 0.0, lax.rsqrt(deg), 0.0)
    s2 = jnp.broadcast_to(s, (BLK, D))
    s2_ref[...] = s2
    u0_ref[...] = s2 * x0_ref[...]


_prologue_call = pl.pallas_call(
    _prologue_tc,
    grid=(NPAD // BLK,),
    in_specs=[
        pl.BlockSpec((BLK, D), lambda i: (i, 0)),
        pl.BlockSpec((BLK, D), lambda i: (i, 0)),
        pl.BlockSpec((BLK, D), lambda i: (i, 0)),
    ],
    out_specs=[
        pl.BlockSpec((BLK, D), lambda i: (i, 0)),
        pl.BlockSpec((BLK, D), lambda i: (i, 0)),
    ],
    out_shape=[
        jax.ShapeDtypeStruct((NPAD, D), jnp.float32),
        jax.ShapeDtypeStruct((NPAD, D), jnp.float32),
    ],
)


def _layer_tc(a0_ref, a1_ref, s2_ref, sm_ref, sm_out, u_out):
    x = s2_ref[...] * (a0_ref[...] + a1_ref[...])
    sm_out[...] = sm_ref[...] + x
    u_out[...] = s2_ref[...] * x


_layer_call = pl.pallas_call(
    _layer_tc,
    grid=(NPAD // BLK,),
    in_specs=[pl.BlockSpec((BLK, D), lambda i: (i, 0))] * 4,
    out_specs=[pl.BlockSpec((BLK, D), lambda i: (i, 0))] * 2,
    out_shape=[
        jax.ShapeDtypeStruct((NPAD, D), jnp.float32),
        jax.ShapeDtypeStruct((NPAD, D), jnp.float32),
    ],
)


def _final_tc(a0_ref, a1_ref, s2_ref, sm_ref, out_ref):
    x = s2_ref[...] * (a0_ref[...] + a1_ref[...])
    out_ref[...] = (sm_ref[...] + x) * 0.25


_final_call = pl.pallas_call(
    _final_tc,
    grid=(NPAD // BLK,),
    in_specs=[pl.BlockSpec((BLK, D), lambda i: (i, 0))] * 4,
    out_specs=pl.BlockSpec((BLK, D), lambda i: (i, 0)),
    out_shape=jax.ShapeDtypeStruct((NPAD, D), jnp.float32),
)


def kernel(embedding_weight, edge_index):
    x0 = embedding_weight.astype(jnp.float32)
    ei = edge_index.astype(jnp.int32)
    pad_ids = (jnp.arange(EPAD - E, dtype=jnp.int32) % (NPAD - N)) + N
    rowp = jnp.concatenate([ei[0], pad_ids]).reshape(NW, CPT, CHUNK)
    colp = jnp.concatenate([ei[1], pad_ids]).reshape(NW, CPT, CHUNK)
    x0p = jnp.pad(x0, ((0, NPAD - N), (0, 0)))
    z128 = jnp.zeros((NPAD, D), jnp.float32)
    ones128 = jnp.ones((CHUNK, D), jnp.float32)

    degp = _deg_call(colp, ones128, z128)
    u, s2 = _prologue_call(degp[0], degp[1], x0p)
    sm = x0p
    out = None
    for layer in range(NLAYERS):
        acc = _prop_call(u, rowp, colp, z128)
        if layer < NLAYERS - 1:
            sm, u = _layer_call(acc[0], acc[1], s2, sm)
        else:
            out = _final_call(acc[0], acc[1], s2, sm)
    return out[:N]


# speedup vs baseline: 16.2528x; 1.3644x over previous
import functools

import jax
import jax.numpy as jnp
from jax import lax
from jax.experimental import pallas as pl
from jax.experimental.pallas import tpu as pltpu
from jax.experimental.pallas import tpu_sc as plsc

N = 10000
D = 128
E = 320000
NLAYERS = 3
NC = 2
NS = 16
NW = NC * NS
CHUNK = 128
CPT = 80
HALF = CPT // 2
EPAD = NW * CPT * CHUNK
NPAD = 10240
RPT = NPAD // NS

_mesh = plsc.VectorSubcoreMesh(
    core_axis_name="c", subcore_axis_name="s", num_cores=NC, num_subcores=NS)


def _deg_body(colp_hbm, ones_hbm, zeros_hbm, out_hbm, col_v, ones_v, acc_sh, sem):
    cid = lax.axis_index("c")
    sid = lax.axis_index("s")
    wid = sid * NC + cid
    r0 = sid * RPT
    pltpu.sync_copy(zeros_hbm.at[pl.ds(r0, RPT)], acc_sh.at[pl.ds(r0, RPT)])
    pltpu.sync_copy(colp_hbm.at[wid], col_v)
    pltpu.sync_copy(ones_hbm, ones_v)
    plsc.subcore_barrier()

    def fire(j, carry):
        pltpu.async_copy(ones_v, acc_sh.at[col_v.at[j]], sem, add=True)
        return carry

    lax.fori_loop(0, CPT, fire, 0)

    def drain(j, carry):
        pltpu.make_async_copy(ones_v, acc_sh.at[col_v.at[j]], sem).wait()
        return carry

    lax.fori_loop(0, CPT, drain, 0)
    plsc.subcore_barrier()
    pltpu.sync_copy(acc_sh.at[pl.ds(r0, RPT)], out_hbm.at[cid, pl.ds(r0, RPT)])


_deg_call = pl.kernel(
    _deg_body,
    out_type=jax.ShapeDtypeStruct((NC, NPAD, D), jnp.float32),
    mesh=_mesh,
    scratch_types=[
        pltpu.VMEM((CPT, CHUNK), jnp.int32),
        pltpu.VMEM((CHUNK, D), jnp.float32),
        pltpu.VMEM_SHARED((NPAD, D), jnp.float32),
        pltpu.SemaphoreType.DMA,
    ],
)


def _prop_body(u_hbm, rowp_hbm, colp_hbm, zeros_hbm, out_hbm,
               row_v, col_v, gbuf, acc_sh, sems):
    cid = lax.axis_index("c")
    sid = lax.axis_index("s")
    wid = sid * NC + cid
    r0 = sid * RPT
    pltpu.sync_copy(zeros_hbm.at[pl.ds(r0, RPT)], acc_sh.at[pl.ds(r0, RPT)])

    def start_gather(j, b):
        pltpu.async_copy(u_hbm.at[row_v.at[j]], gbuf.at[b], sems.at[b])

    def wait_gather(j, b):
        pltpu.make_async_copy(u_hbm.at[row_v.at[j]], gbuf.at[b], sems.at[b]).wait()

    for h in range(CPT // HALF):
        pltpu.sync_copy(rowp_hbm.at[wid, pl.ds(h * HALF, HALF)], row_v)
        pltpu.sync_copy(colp_hbm.at[wid, pl.ds(h * HALF, HALF)], col_v)
        if h == 0:
            plsc.subcore_barrier()
        start_gather(0, 0)

        def body(j, carry):
            b = lax.rem(j, 2)

            @pl.when(j + 1 < HALF)
            def _():
                start_gather(j + 1, 1 - b)

            wait_gather(j, b)
            pltpu.sync_copy(gbuf.at[b], acc_sh.at[col_v.at[j]], add=True)
            return carry

        lax.fori_loop(0, HALF, body, 0)

    plsc.subcore_barrier()
    pltpu.sync_copy(acc_sh.at[pl.ds(r0, RPT)], out_hbm.at[cid, pl.ds(r0, RPT)])


_prop_call = pl.kernel(
    _prop_body,
    out_type=jax.ShapeDtypeStruct((NC, NPAD, D), jnp.float32),
    mesh=_mesh,
    scratch_types=[
        pltpu.VMEM((HALF, CHUNK), jnp.int32),
        pltpu.VMEM((HALF, CHUNK), jnp.int32),
        pltpu.VMEM((2, CHUNK, D), jnp.float32),
        pltpu.VMEM_SHARED((NPAD, D), jnp.float32),
        pltpu.SemaphoreType.DMA((2,)),
    ],
)


BLK = 512


def _prologue_tc(d0_ref, d1_ref, x0_ref, u0_ref, s2_ref):
    deg = d0_ref[:, 0:1] + d1_ref[:, 0:1]
    s = jnp.where(deg > 0.0, lax.rsqrt(deg), 0.0)
    s2 = jnp.broadcast_to(s, (BLK, D))
    s2_ref[...] = s2
    u0_ref[...] = s2 * x0_ref[...]


_prologue_call = pl.pallas_call(
    _prologue_tc,
    grid=(NPAD // BLK,),
    in_specs=[
        pl.BlockSpec((BLK, D), lambda i: (i, 0)),
        pl.BlockSpec((BLK, D), lambda i: (i, 0)),
        pl.BlockSpec((BLK, D), lambda i: (i, 0)),
    ],
    out_specs=[
        pl.BlockSpec((BLK, D), lambda i: (i, 0)),
        pl.BlockSpec((BLK, D), lambda i: (i, 0)),
    ],
    out_shape=[
        jax.ShapeDtypeStruct((NPAD, D), jnp.float32),
        jax.ShapeDtypeStruct((NPAD, D), jnp.float32),
    ],
)


def _layer_tc(a0_ref, a1_ref, s2_ref, sm_ref, sm_out, u_out):
    x = s2_ref[...] * (a0_ref[...] + a1_ref[...])
    sm_out[...] = sm_ref[...] + x
    u_out[...] = s2_ref[...] * x


_layer_call = pl.pallas_call(
    _layer_tc,
    grid=(NPAD // BLK,),
    in_specs=[pl.BlockSpec((BLK, D), lambda i: (i, 0))] * 4,
    out_specs=[pl.BlockSpec((BLK, D), lambda i: (i, 0))] * 2,
    out_shape=[
        jax.ShapeDtypeStruct((NPAD, D), jnp.float32),
        jax.ShapeDtypeStruct((NPAD, D), jnp.float32),
    ],
)


def _final_tc(a0_ref, a1_ref, s2_ref, sm_ref, out_ref):
    x = s2_ref[...] * (a0_ref[...] + a1_ref[...])
    out_ref[...] = (sm_ref[...] + x) * 0.25


_final_call = pl.pallas_call(
    _final_tc,
    grid=(NPAD // BLK,),
    in_specs=[pl.BlockSpec((BLK, D), lambda i: (i, 0))] * 4,
    out_specs=pl.BlockSpec((BLK, D), lambda i: (i, 0)),
    out_shape=jax.ShapeDtypeStruct((NPAD, D), jnp.float32),
)


def kernel(embedding_weight, edge_index):
    x0 = embedding_weight.astype(jnp.float32)
    ei = edge_index.astype(jnp.int32)
    pad_ids = (jnp.arange(EPAD - E, dtype=jnp.int32) % (NPAD - N)) + N
    rowp = jnp.concatenate([ei[0], pad_ids]).reshape(NW, CPT, CHUNK)
    colp = jnp.concatenate([ei[1], pad_ids]).reshape(NW, CPT, CHUNK)
    x0p = jnp.pad(x0, ((0, NPAD - N), (0, 0)))
    z128 = jnp.zeros((NPAD, D), jnp.float32)
    ones128 = jnp.ones((CHUNK, D), jnp.float32)

    degp = _deg_call(colp, ones128, z128)
    u, s2 = _prologue_call(degp[0], degp[1], x0p)
    sm = x0p
    out = None
    for layer in range(NLAYERS):
        acc = _prop_call(u, rowp, colp, z128)
        if layer < NLAYERS - 1:
            sm, u = _layer_call(acc[0], acc[1], s2, sm)
        else:
            out = _final_call(acc[0], acc[1], s2, sm)
    return out[:N]


# speedup vs baseline: 17.5148x; 1.0776x over previous
import functools

import jax
import jax.numpy as jnp
from jax import lax
from jax.experimental import pallas as pl
from jax.experimental.pallas import tpu as pltpu
from jax.experimental.pallas import tpu_sc as plsc

N = 10000
D = 128
E = 320000
NLAYERS = 3
NC = 2
NS = 16
NW = NC * NS
CHUNK = 120
CPT = 84
EPAD = NW * CPT * CHUNK
NPAD = 10112
RPT = NPAD // NS

_mesh = plsc.VectorSubcoreMesh(
    core_axis_name="c", subcore_axis_name="s", num_cores=NC, num_subcores=NS)


def _ign(ref):
    return plsc.Indices(ref, ignored_value=-1)


def _deg_body(colp_hbm, ones_hbm, zeros_hbm, out_hbm, col_v, ones_v, acc_sh, sem):
    cid = lax.axis_index("c")
    sid = lax.axis_index("s")
    wid = sid * NC + cid
    r0 = sid * RPT
    pltpu.sync_copy(zeros_hbm.at[pl.ds(r0, RPT)], acc_sh.at[pl.ds(r0, RPT)])
    pltpu.sync_copy(colp_hbm.at[wid], col_v)
    pltpu.sync_copy(ones_hbm, ones_v)
    plsc.subcore_barrier()

    def fire(j, carry):
        pltpu.async_copy(ones_v, acc_sh.at[_ign(col_v.at[j])], sem, add=True)
        return carry

    lax.fori_loop(0, CPT, fire, 0)

    def drain(j, carry):
        pltpu.make_async_copy(ones_v, acc_sh.at[_ign(col_v.at[j])], sem).wait()
        return carry

    lax.fori_loop(0, CPT, drain, 0)
    plsc.subcore_barrier()
    pltpu.sync_copy(acc_sh.at[pl.ds(r0, RPT)], out_hbm.at[cid, pl.ds(r0, RPT)])


_deg_call = pl.kernel(
    _deg_body,
    out_type=jax.ShapeDtypeStruct((NC, NPAD, D), jnp.float32),
    mesh=_mesh,
    scratch_types=[
        pltpu.VMEM((CPT, CHUNK), jnp.int32),
        pltpu.VMEM((CHUNK, D), jnp.float32),
        pltpu.VMEM_SHARED((NPAD, D), jnp.float32),
        pltpu.SemaphoreType.DMA,
    ],
)


def _prop_body(u_hbm, rowp_hbm, colp_hbm, zeros_hbm, out_hbm,
               idx_v, gbuf, acc_sh, rsem, csem, gsem):
    cid = lax.axis_index("c")
    sid = lax.axis_index("s")
    wid = sid * NC + cid
    r0 = sid * RPT
    pltpu.sync_copy(zeros_hbm.at[pl.ds(r0, RPT)], acc_sh.at[pl.ds(r0, RPT)])
    plsc.subcore_barrier()

    def start_idx(j):
        b = lax.rem(j, 3)
        pltpu.async_copy(rowp_hbm.at[wid, j], idx_v.at[b], rsem.at[b])
        pltpu.async_copy(colp_hbm.at[wid, j], idx_v.at[3 + b], csem.at[b])

    def wait_idx(j):
        b = lax.rem(j, 3)
        pltpu.make_async_copy(rowp_hbm.at[wid, j], idx_v.at[b], rsem.at[b]).wait()
        pltpu.make_async_copy(colp_hbm.at[wid, j], idx_v.at[3 + b], csem.at[b]).wait()

    def start_gather(j):
        b = lax.rem(j, 3)
        pltpu.async_copy(u_hbm.at[_ign(idx_v.at[b])], gbuf.at[b], gsem.at[b])

    def wait_gather(j):
        b = lax.rem(j, 3)
        pltpu.make_async_copy(
            u_hbm.at[_ign(idx_v.at[b])], gbuf.at[b], gsem.at[b]).wait()

    def scatter(j):
        b = lax.rem(j, 3)
        pltpu.sync_copy(gbuf.at[b], acc_sh.at[_ign(idx_v.at[3 + b])], add=True)

    start_idx(0)
    start_idx(1)
    wait_idx(0)
    start_gather(0)

    def body(j, carry):
        @pl.when(j + 2 < CPT)
        def _():
            start_idx(j + 2)

        @pl.when(j + 1 < CPT)
        def _():
            wait_idx(j + 1)
            start_gather(j + 1)

        wait_gather(j)
        scatter(j)
        return carry

    lax.fori_loop(0, CPT, body, 0)
    plsc.subcore_barrier()
    pltpu.sync_copy(acc_sh.at[pl.ds(r0, RPT)], out_hbm.at[cid, pl.ds(r0, RPT)])


_prop_call = pl.kernel(
    _prop_body,
    out_type=jax.ShapeDtypeStruct((NC, NPAD, D), jnp.float32),
    mesh=_mesh,
    scratch_types=[
        pltpu.VMEM((6, CHUNK), jnp.int32),
        pltpu.VMEM((3, CHUNK, D), jnp.float32),
        pltpu.VMEM_SHARED((NPAD, D), jnp.float32),
        pltpu.SemaphoreType.DMA((3,)),
        pltpu.SemaphoreType.DMA((3,)),
        pltpu.SemaphoreType.DMA((3,)),
    ],
)


BLK = 1264


def _prologue_tc(d0_ref, d1_ref, x0_ref, u0_ref, s2_ref):
    deg = d0_ref[:, 0:1] + d1_ref[:, 0:1]
    s = jnp.where(deg > 0.0, lax.rsqrt(deg), 0.0)
    s2 = jnp.broadcast_to(s, (BLK, D))
    s2_ref[...] = s2
    u0_ref[...] = s2 * x0_ref[...]


_prologue_call = pl.pallas_call(
    _prologue_tc,
    grid=(NPAD // BLK,),
    in_specs=[
        pl.BlockSpec((BLK, D), lambda i: (i, 0)),
        pl.BlockSpec((BLK, D), lambda i: (i, 0)),
        pl.BlockSpec((BLK, D), lambda i: (i, 0)),
    ],
    out_specs=[
        pl.BlockSpec((BLK, D), lambda i: (i, 0)),
        pl.BlockSpec((BLK, D), lambda i: (i, 0)),
    ],
    out_shape=[
        jax.ShapeDtypeStruct((NPAD, D), jnp.float32),
        jax.ShapeDtypeStruct((NPAD, D), jnp.float32),
    ],
)


def _layer_tc(a0_ref, a1_ref, s2_ref, sm_ref, sm_out, u_out):
    x = s2_ref[...] * (a0_ref[...] + a1_ref[...])
    sm_out[...] = sm_ref[...] + x
    u_out[...] = s2_ref[...] * x


_layer_call = pl.pallas_call(
    _layer_tc,
    grid=(NPAD // BLK,),
    in_specs=[pl.BlockSpec((BLK, D), lambda i: (i, 0))] * 4,
    out_specs=[pl.BlockSpec((BLK, D), lambda i: (i, 0))] * 2,
    out_shape=[
        jax.ShapeDtypeStruct((NPAD, D), jnp.float32),
        jax.ShapeDtypeStruct((NPAD, D), jnp.float32),
    ],
)


def _final_tc(a0_ref, a1_ref, s2_ref, sm_ref, out_ref):
    x = s2_ref[...] * (a0_ref[...] + a1_ref[...])
    out_ref[...] = (sm_ref[...] + x) * 0.25


_final_call = pl.pallas_call(
    _final_tc,
    grid=(NPAD // BLK,),
    in_specs=[pl.BlockSpec((BLK, D), lambda i: (i, 0))] * 4,
    out_specs=pl.BlockSpec((BLK, D), lambda i: (i, 0)),
    out_shape=jax.ShapeDtypeStruct((NPAD, D), jnp.float32),
)


def kernel(embedding_weight, edge_index):
    x0 = embedding_weight.astype(jnp.float32)
    ei = edge_index.astype(jnp.int32)
    pad_ids = jnp.full((EPAD - E,), -1, jnp.int32)
    rowp = jnp.concatenate([ei[0], pad_ids]).reshape(NW, CPT, CHUNK)
    colp = jnp.concatenate([ei[1], pad_ids]).reshape(NW, CPT, CHUNK)
    x0p = jnp.pad(x0, ((0, NPAD - N), (0, 0)))
    z128 = jnp.zeros((NPAD, D), jnp.float32)
    ones128 = jnp.ones((CHUNK, D), jnp.float32)

    degp = _deg_call(colp, ones128, z128)
    u, s2 = _prologue_call(degp[0], degp[1], x0p)
    sm = x0p
    out = None
    for layer in range(NLAYERS):
        acc = _prop_call(u, rowp, colp, z128)
        if layer < NLAYERS - 1:
            sm, u = _layer_call(acc[0], acc[1], s2, sm)
        else:
            out = _final_call(acc[0], acc[1], s2, sm)
    return out[:N]


# speedup vs baseline: 18.4700x; 1.0545x over previous
import functools

import jax
import jax.numpy as jnp
from jax import lax
from jax.experimental import pallas as pl
from jax.experimental.pallas import tpu as pltpu
from jax.experimental.pallas import tpu_sc as plsc

N = 10000
D = 128
E = 320000
NLAYERS = 3
NC = 2
NS = 16
NW = NC * NS
CHUNK = 120
CPT = 84
EPAD = NW * CPT * CHUNK
NPAD = 10112
RPT = NPAD // NS

_mesh = plsc.VectorSubcoreMesh(
    core_axis_name="c", subcore_axis_name="s", num_cores=NC, num_subcores=NS)


def _ign(ref):
    return plsc.Indices(ref, ignored_value=-1)


def _deg_body(colp_hbm, ones_hbm, zeros_hbm, out_hbm, col_v, ones_v, acc_sh, sem):
    cid = lax.axis_index("c")
    sid = lax.axis_index("s")
    wid = sid * NC + cid
    r0 = sid * RPT
    pltpu.sync_copy(zeros_hbm.at[pl.ds(r0, RPT)], acc_sh.at[pl.ds(r0, RPT)])
    pltpu.sync_copy(colp_hbm.at[wid], col_v)
    pltpu.sync_copy(ones_hbm, ones_v)
    plsc.subcore_barrier()

    def fire(j, carry):
        pltpu.async_copy(ones_v, acc_sh.at[_ign(col_v.at[j])], sem, add=True)
        return carry

    lax.fori_loop(0, CPT, fire, 0)

    def drain(j, carry):
        pltpu.make_async_copy(ones_v, acc_sh.at[_ign(col_v.at[j])], sem).wait()
        return carry

    lax.fori_loop(0, CPT, drain, 0)
    plsc.subcore_barrier()
    pltpu.sync_copy(acc_sh.at[pl.ds(r0, RPT)], out_hbm.at[cid, pl.ds(r0, RPT)])


_deg_call = pl.kernel(
    _deg_body,
    out_type=jax.ShapeDtypeStruct((NC, NPAD, D), jnp.float32),
    mesh=_mesh,
    scratch_types=[
        pltpu.VMEM((CPT, CHUNK), jnp.int32),
        pltpu.VMEM((CHUNK, D), jnp.float32),
        pltpu.VMEM_SHARED((NPAD, D), jnp.float32),
        pltpu.SemaphoreType.DMA,
    ],
)


def _prop_body(u_hbm, rowp_hbm, colp_hbm, zeros_hbm, out_hbm,
               idx_v, gbuf, acc_sh, rsem, csem, gsem, ssem):
    cid = lax.axis_index("c")
    sid = lax.axis_index("s")
    wid = sid * NC + cid
    r0 = sid * RPT
    pltpu.sync_copy(zeros_hbm.at[pl.ds(r0, RPT)], acc_sh.at[pl.ds(r0, RPT)])
    plsc.subcore_barrier()

    def start_idx(j):
        b = lax.rem(j, 4)
        pltpu.async_copy(rowp_hbm.at[wid, j], idx_v.at[b], rsem.at[b])
        pltpu.async_copy(colp_hbm.at[wid, j], idx_v.at[4 + b], csem.at[b])

    def wait_idx(j):
        b = lax.rem(j, 4)
        pltpu.make_async_copy(rowp_hbm.at[wid, j], idx_v.at[b], rsem.at[b]).wait()
        pltpu.make_async_copy(colp_hbm.at[wid, j], idx_v.at[4 + b], csem.at[b]).wait()

    def start_gather(j):
        b = lax.rem(j, 3)
        pltpu.async_copy(u_hbm.at[_ign(idx_v.at[lax.rem(j, 4)])],
                         gbuf.at[b], gsem.at[b])

    def wait_gather(j):
        b = lax.rem(j, 3)
        pltpu.make_async_copy(u_hbm.at[_ign(idx_v.at[lax.rem(j, 4)])],
                              gbuf.at[b], gsem.at[b]).wait()

    def start_scatter(j):
        b = lax.rem(j, 3)
        pltpu.async_copy(gbuf.at[b], acc_sh.at[_ign(idx_v.at[4 + lax.rem(j, 4)])],
                         ssem.at[b], add=True)

    def wait_scatter(j):
        b = lax.rem(j, 3)
        pltpu.make_async_copy(gbuf.at[b],
                              acc_sh.at[_ign(idx_v.at[4 + lax.rem(j, 4)])],
                              ssem.at[b]).wait()

    start_idx(0)
    start_idx(1)
    wait_idx(0)
    start_gather(0)

    def body(j, carry):
        @pl.when(j >= 2)
        def _():
            wait_scatter(j - 2)

        @pl.when(j + 2 < CPT)
        def _():
            start_idx(j + 2)

        @pl.when(j + 1 < CPT)
        def _():
            wait_idx(j + 1)
            start_gather(j + 1)

        wait_gather(j)
        start_scatter(j)
        return carry

    lax.fori_loop(0, CPT, body, 0)
    wait_scatter(CPT - 2)
    wait_scatter(CPT - 1)
    plsc.subcore_barrier()
    pltpu.sync_copy(acc_sh.at[pl.ds(r0, RPT)], out_hbm.at[cid, pl.ds(r0, RPT)])


_prop_call = pl.kernel(
    _prop_body,
    out_type=jax.ShapeDtypeStruct((NC, NPAD, D), jnp.float32),
    mesh=_mesh,
    scratch_types=[
        pltpu.VMEM((8, CHUNK), jnp.int32),
        pltpu.VMEM((3, CHUNK, D), jnp.float32),
        pltpu.VMEM_SHARED((NPAD, D), jnp.float32),
        pltpu.SemaphoreType.DMA((4,)),
        pltpu.SemaphoreType.DMA((4,)),
        pltpu.SemaphoreType.DMA((3,)),
        pltpu.SemaphoreType.DMA((3,)),
    ],
)


BLK = 1264


def _prologue_tc(d0_ref, d1_ref, x0_ref, u0_ref, s2_ref):
    deg = d0_ref[:, 0:1] + d1_ref[:, 0:1]
    s = jnp.where(deg > 0.0, lax.rsqrt(deg), 0.0)
    s2 = jnp.broadcast_to(s, (BLK, D))
    s2_ref[...] = s2
    u0_ref[...] = s2 * x0_ref[...]


_prologue_call = pl.pallas_call(
    _prologue_tc,
    grid=(NPAD // BLK,),
    in_specs=[
        pl.BlockSpec((BLK, D), lambda i: (i, 0)),
        pl.BlockSpec((BLK, D), lambda i: (i, 0)),
        pl.BlockSpec((BLK, D), lambda i: (i, 0)),
    ],
    out_specs=[
        pl.BlockSpec((BLK, D), lambda i: (i, 0)),
        pl.BlockSpec((BLK, D), lambda i: (i, 0)),
    ],
    out_shape=[
        jax.ShapeDtypeStruct((NPAD, D), jnp.float32),
        jax.ShapeDtypeStruct((NPAD, D), jnp.float32),
    ],
)


def _layer_tc(a0_ref, a1_ref, s2_ref, sm_ref, sm_out, u_out):
    x = s2_ref[...] * (a0_ref[...] + a1_ref[...])
    sm_out[...] = sm_ref[...] + x
    u_out[...] = s2_ref[...] * x


_layer_call = pl.pallas_call(
    _layer_tc,
    grid=(NPAD // BLK,),
    in_specs=[pl.BlockSpec((BLK, D), lambda i: (i, 0))] * 4,
    out_specs=[pl.BlockSpec((BLK, D), lambda i: (i, 0))] * 2,
    out_shape=[
        jax.ShapeDtypeStruct((NPAD, D), jnp.float32),
        jax.ShapeDtypeStruct((NPAD, D), jnp.float32),
    ],
)


def _final_tc(a0_ref, a1_ref, s2_ref, sm_ref, out_ref):
    x = s2_ref[...] * (a0_ref[...] + a1_ref[...])
    out_ref[...] = (sm_ref[...] + x) * 0.25


_final_call = pl.pallas_call(
    _final_tc,
    grid=(NPAD // BLK,),
    in_specs=[pl.BlockSpec((BLK, D), lambda i: (i, 0))] * 4,
    out_specs=pl.BlockSpec((BLK, D), lambda i: (i, 0)),
    out_shape=jax.ShapeDtypeStruct((NPAD, D), jnp.float32),
)


def kernel(embedding_weight, edge_index):
    x0 = embedding_weight.astype(jnp.float32)
    ei = edge_index.astype(jnp.int32)
    pad_ids = jnp.full((EPAD - E,), -1, jnp.int32)
    rowp = jnp.concatenate([ei[0], pad_ids]).reshape(NW, CPT, CHUNK)
    colp = jnp.concatenate([ei[1], pad_ids]).reshape(NW, CPT, CHUNK)
    x0p = jnp.pad(x0, ((0, NPAD - N), (0, 0)))
    z128 = jnp.zeros((NPAD, D), jnp.float32)
    ones128 = jnp.ones((CHUNK, D), jnp.float32)

    degp = _deg_call(colp, ones128, z128)
    u, s2 = _prologue_call(degp[0], degp[1], x0p)
    sm = x0p
    out = None
    for layer in range(NLAYERS):
        acc = _prop_call(u, rowp, colp, z128)
        if layer < NLAYERS - 1:
            sm, u = _layer_call(acc[0], acc[1], s2, sm)
        else:
            out = _final_call(acc[0], acc[1], s2, sm)
    return out[:N]
